# baseline (speedup 1.0000x reference)
import functools

import numpy as np
import jax
import jax.numpy as jnp
from jax import lax
from jax.experimental import pallas as pl
from jax.experimental.pallas import tpu as pltpu
from jax.experimental.pallas import tpu_sc as plsc

LN_EPS = 1e-3
L = 16


def _posenc_np(max_len, d):
    pos = np.arange(max_len)[:, None]
    i = np.arange(d)[None, :]
    ang = pos * (1.0 / np.power(10000, 2 * (i // 2) / np.float32(d)))
    ang[:, 0::2] = np.sin(ang[:, 0::2])
    ang[:, 1::2] = np.cos(ang[:, 1::2])
    return ang.astype(np.float32)


def _rsqrt_vec(v):
    i = lax.bitcast_convert_type(v, jnp.int32)
    i = jnp.int32(0x5F3759DF) - lax.shift_right_logical(i, jnp.int32(1))
    y = lax.bitcast_convert_type(i, jnp.float32)
    for _ in range(2):
        y = y * (1.5 - 0.5 * v * y * y)
    return y


@functools.lru_cache(maxsize=None)
def _build(NT, E, S, CHUNK):
    info = plsc.get_sparse_core_info()
    NC, NS = info.num_cores, info.num_subcores
    NW = NC * NS
    TPW = NT // NW
    NCHUNK = TPW // CHUNK
    assert NCHUNK % 2 == 0
    KD = E // L
    NG = CHUNK // L
    mesh = plsc.VectorSubcoreMesh(core_axis_name="c", subcore_axis_name="s")

    @functools.partial(
        pl.kernel,
        mesh=mesh,
        compiler_params=pltpu.CompilerParams(
            use_tc_tiling_on_sc=False, needs_layout_passes=False),
        out_type=jax.ShapeDtypeStruct((NT, E), jnp.float32),
        scratch_types=[
            pltpu.VMEM((2, CHUNK), jnp.int32),
            pltpu.VMEM((CHUNK, E), jnp.float32),
            pltpu.VMEM((CHUNK, E), jnp.float32),
            pltpu.VMEM((L, E), jnp.float32),
            pltpu.VMEM((2 * L,), jnp.float32),
            pltpu.VMEM((E, S + L), jnp.float32),
            pltpu.VMEM((2, E), jnp.float32),
            [pltpu.SemaphoreType.DMA] * 6,
        ],
    )
    def k(ids_hbm, table_hbm, post_hbm, gb_hbm, out_hbm,
          ids_v, rows0, rows1, xbuf, mr_v, post_v, gb_v, sems):
        cid = lax.axis_index("c")
        sid = lax.axis_index("s")
        wid = sid * NC + cid
        t_base = wid * TPW
        rows = [rows0, rows1]
        sem_ids = [sems[0], sems[1]]
        sem_g = [sems[2], sems[3]]
        sem_out = [sems[4], sems[5]]

        pltpu.sync_copy(post_hbm, post_v)
        pltpu.sync_copy(gb_hbm, gb_v)
        gs = [gb_v[0, pl.ds(kk * L, L)] for kk in range(KD)]
        bs = [gb_v[1, pl.ds(kk * L, L)] for kk in range(KD)]
        lanes = lax.iota(jnp.int32, L)

        def gather_of(par):
            return pltpu.make_async_copy(
                table_hbm.at[ids_v.at[par]], rows[par], sem_g[par])

        def out_of(ci, par):
            t0 = t_base + ci * CHUNK
            return pltpu.make_async_copy(
                rows[par], out_hbm.at[pl.ds(t0, CHUNK)], sem_out[par])

        def ids_of(ci, par):
            t0 = t_base + ci * CHUNK
            return pltpu.make_async_copy(
                ids_hbm.at[pl.ds(t0, CHUNK)], ids_v.at[par], sem_ids[par])

        def compute(t0, par):
            rv = rows[par]

            def grp(g, _):
                base = g * L
                tok = base + lanes
                s0 = lax.rem(t0 + base, S)
                zv = jnp.zeros((L,), jnp.float32)

                @plsc.parallel_loop(0, E, step=2, unroll=4,
                                    carry=(zv, zv, zv, zv))
                def acc_loop(d, accs):
                    a1, a2, b1, b2 = accs
                    col = jnp.full((L,), d, jnp.int32)
                    x = plsc.load_gather(rv, [tok, col]) + post_v[d, pl.ds(s0, L)]
                    plsc.store_scatter(xbuf, [lanes, col], x)
                    col2 = col + 1
                    y = plsc.load_gather(rv, [tok, col2]) + post_v[d + 1, pl.ds(s0, L)]
                    plsc.store_scatter(xbuf, [lanes, col2], y)
                    return (a1 + x, a2 + x * x, b1 + y, b2 + y * y)

                a1, a2, b1, b2 = acc_loop
                mean = (a1 + b1) * (1.0 / E)
                var = (a2 + b2) * (1.0 / E) - mean * mean
                rstd = _rsqrt_vec(var + LN_EPS)
                mr_v[pl.ds(0, L)] = mean
                mr_v[pl.ds(L, L)] = rstd

                @plsc.parallel_loop(0, L, unroll=4)
                def norm_loop(u):
                    um = jnp.full((L,), u, jnp.int32)
                    m = plsc.load_gather(mr_v, [um])
                    r = plsc.load_gather(mr_v, [um + L])
                    ii = base + u
                    for kk in range(KD):
                        xv = xbuf[u, pl.ds(kk * L, L)]
                        rv[ii, pl.ds(kk * L, L)] = (xv - m) * r * gs[kk] + bs[kk]

                return 0

            lax.fori_loop(0, NG, grp, 0)

        c0 = ids_of(0, 0)
        c0.start()
        c0.wait()
        gather_of(0).start()
        ids_of(1, 1).start()

        def pair_body(cp, _):
            for par in range(2):
                ci = cp * 2 + par
                t0 = t_base + ci * CHUNK

                @pl.when(ci >= 1)
                def _():
                    out_of(ci - 1, 1 - par).wait()

                @pl.when(ci + 1 < NCHUNK)
                def _():
                    ids_of(ci + 1, 1 - par).wait()
                    gather_of(1 - par).start()

                gather_of(par).wait()
                compute(t0, par)
                out_of(ci, par).start()

                @pl.when(ci + 2 < NCHUNK)
                def _():
                    ids_of(ci + 2, par).start()
            return 0

        lax.fori_loop(0, NCHUNK // 2, pair_body, 0)
        out_of(NCHUNK - 1, (NCHUNK - 1) % 2).wait()

    return k


def kernel(input_ids, token_type_ids, table, gamma, beta):
    B, S = input_ids.shape
    V, E = table.shape
    NT = B * S
    ids = input_ids.reshape(NT).astype(jnp.int32)
    pos = _posenc_np(S, E)
    post = jnp.asarray(
        np.concatenate([pos.T, pos.T[:, :L]], axis=1))
    gb = jnp.stack([gamma, beta])
    out = _build(NT, E, S, 512)(ids, table, post, gb)
    return out.reshape(B, S, E)
